# baseline (speedup 1.0000x reference)
import functools

import jax
import jax.numpy as jnp
from jax import lax
from jax.experimental import pallas as pl
from jax.experimental.pallas import tpu as pltpu
from jax.experimental.pallas import tpu_sc as plsc

D = 128
LANES = 16
N_SUBCORES = 16
CHUNK = 128


def _tables_body(x_ref, w1_ref, w2_ref, be_ref, p_ref, q_ref):
    xb = x_ref[...]
    p_ref[...] = (
        jnp.dot(xb, w1_ref[...], preferred_element_type=jnp.float32)
        + be_ref[...]
    )
    q_ref[...] = jnp.dot(xb, w2_ref[...], preferred_element_type=jnp.float32)


def _edge_mm_body(ea_ref, w3_ref, t_ref):
    t_ref[...] = jnp.dot(
        ea_ref[...], w3_ref[...], preferred_element_type=jnp.float32
    )


def _node_body(x_ref, agg_ref, wn_ref, bn_ref, o_ref):
    acc = jnp.dot(x_ref[...], wn_ref[:D, :], preferred_element_type=jnp.float32)
    acc += jnp.dot(agg_ref[...], wn_ref[D:, :], preferred_element_type=jnp.float32)
    o_ref[...] = jnp.maximum(acc + bn_ref[...], 0.0)


def _row_block(bm):
    return pl.BlockSpec((bm, D), lambda i: (i, 0))


def _full_block(shape):
    return pl.BlockSpec(shape, lambda i: tuple(0 for _ in shape))


def _make_sc_edge_kernel(n_nodes, n_edges, n_batch):
    chunks_per_batch = n_edges // CHUNK
    n_iters = (chunks_per_batch + N_SUBCORES - 1) // N_SUBCORES
    rows_per_tile = (n_nodes // N_SUBCORES) // 8 * 8
    rows_rem = n_nodes - rows_per_tile * N_SUBCORES
    groups = D // LANES

    mesh = plsc.VectorSubcoreMesh(core_axis_name="c", subcore_axis_name="s")

    @functools.partial(
        pl.kernel,
        out_type=[
            jax.ShapeDtypeStruct((n_batch * n_edges, D), jnp.float32),
            jax.ShapeDtypeStruct((n_batch * n_nodes, D), jnp.float32),
        ],
        mesh=mesh,
        scratch_types=[
            pltpu.VMEM((CHUNK, D), jnp.float32),
            pltpu.VMEM((CHUNK, D), jnp.float32),
            pltpu.VMEM((CHUNK, D), jnp.float32),
            pltpu.VMEM((CHUNK,), jnp.int32),
            pltpu.VMEM((CHUNK,), jnp.int32),
            pltpu.VMEM((1, CHUNK), jnp.int32),
            pltpu.VMEM_SHARED((n_nodes, D), jnp.float32),
            pltpu.SemaphoreType.DMA,
        ],
    )
    def sc_kernel(t_hbm, p_hbm, q_hbm, si_hbm, di_hbm, draw_hbm,
                  m_hbm, agg_hbm, tb, g1, g2, sib, dib, scb, agg_sp, sem):
        b = lax.axis_index("c")
        s = lax.axis_index("s")

        @plsc.parallel_loop(0, CHUNK, unroll=4)
        def _(r):
            for k in range(groups):
                tb[r, pl.ds(k * LANES, LANES)] = jnp.zeros((LANES,), jnp.float32)
        zrows = 0
        while zrows < rows_per_tile:
            step = min(CHUNK, rows_per_tile - zrows)
            pltpu.sync_copy(
                tb.at[pl.ds(0, step)],
                agg_sp.at[pl.ds(s * rows_per_tile + zrows, step)],
            )
            zrows += step
        if rows_rem:
            @pl.when(s == N_SUBCORES - 1)
            def _():
                pltpu.sync_copy(
                    tb.at[pl.ds(0, rows_rem)],
                    agg_sp.at[pl.ds(rows_per_tile * N_SUBCORES, rows_rem)],
                )
        plsc.subcore_barrier()

        def chunk_body(t, carry):
            jloc = s + t * N_SUBCORES

            @pl.when(jloc < chunks_per_batch)
            def _():
                j = b * chunks_per_batch + jloc
                row0 = j * CHUNK
                pltpu.sync_copy(t_hbm.at[pl.ds(row0, CHUNK)], tb)
                pltpu.sync_copy(si_hbm.at[pl.ds(row0, CHUNK)], sib)
                pltpu.sync_copy(di_hbm.at[pl.ds(row0, CHUNK)], dib)
                pltpu.sync_copy(draw_hbm.at[pl.ds(row0, CHUNK)], scb.at[0])
                pltpu.async_copy(p_hbm.at[sib], g1, sem).wait()
                pltpu.async_copy(q_hbm.at[dib], g2, sem).wait()

                @plsc.parallel_loop(0, CHUNK, unroll=4)
                def _(r):
                    for k in range(groups):
                        sl = pl.ds(k * LANES, LANES)
                        tb[r, sl] = jnp.maximum(
                            tb[r, sl] + g1[r, sl] + g2[r, sl], 0.0
                        )

                pltpu.sync_copy(tb, m_hbm.at[pl.ds(row0, CHUNK)])
                pltpu.sync_copy(tb, agg_sp.at[scb.at[0]], add=True)
            return carry
        lax.fori_loop(0, n_iters, chunk_body, None)

        plsc.subcore_barrier()
        pltpu.sync_copy(
            agg_sp.at[pl.ds(s * rows_per_tile, rows_per_tile)],
            agg_hbm.at[pl.ds(b * n_nodes + s * rows_per_tile, rows_per_tile)],
        )
        if rows_rem:
            @pl.when(s == N_SUBCORES - 1)
            def _():
                base = rows_per_tile * N_SUBCORES
                pltpu.sync_copy(
                    agg_sp.at[pl.ds(base, rows_rem)],
                    agg_hbm.at[pl.ds(b * n_nodes + base, rows_rem)],
                )

    return sc_kernel


def kernel(x, edge_index, edge_attr, W_edge, b_edge, W_node, b_node):
    n_batch, n_nodes, d = x.shape
    n_edges = edge_index.shape[1]
    assert d == D and n_edges % CHUNK == 0 and n_nodes % N_SUBCORES == 0

    x2 = x.reshape(n_batch * n_nodes, D)
    ea2 = edge_attr.reshape(n_batch * n_edges, D)
    w1 = W_edge[:D]
    w2 = W_edge[D:2 * D]
    w3 = W_edge[2 * D:]
    be = b_edge.reshape(1, D)
    bn = b_node.reshape(1, D)

    src = edge_index[0].astype(jnp.int32)
    dst = edge_index[1].astype(jnp.int32)
    boff = (jnp.arange(n_batch, dtype=jnp.int32) * n_nodes)[:, None]
    src_adj = (src[None, :] + boff).reshape(-1)
    dst_adj = (dst[None, :] + boff).reshape(-1)
    dst_raw = jnp.broadcast_to(dst, (n_batch, n_edges)).reshape(-1)

    bm = 2000
    p2, q2 = pl.pallas_call(
        _tables_body,
        grid=(n_batch * n_nodes // bm,),
        in_specs=[_row_block(bm), _full_block((D, D)), _full_block((D, D)),
                  _full_block((1, D))],
        out_specs=[_row_block(bm), _row_block(bm)],
        out_shape=[jax.ShapeDtypeStruct((n_batch * n_nodes, D), jnp.float32)] * 2,
    )(x2, w1, w2, be)

    bme = 2000
    t2 = pl.pallas_call(
        _edge_mm_body,
        grid=(n_batch * n_edges // bme,),
        in_specs=[_row_block(bme), _full_block((D, D))],
        out_specs=_row_block(bme),
        out_shape=jax.ShapeDtypeStruct((n_batch * n_edges, D), jnp.float32),
    )(ea2, w3)

    sc = _make_sc_edge_kernel(n_nodes, n_edges, n_batch)
    m2, agg2 = sc(t2, p2, q2, src_adj, dst_adj, dst_raw)

    new_x2 = pl.pallas_call(
        _node_body,
        grid=(n_batch * n_nodes // bm,),
        in_specs=[_row_block(bm), _row_block(bm), _full_block((2 * D, D)),
                  _full_block((1, D))],
        out_specs=_row_block(bm),
        out_shape=jax.ShapeDtypeStruct((n_batch * n_nodes, D), jnp.float32),
    )(x2, agg2, W_node, bn)

    return (new_x2.reshape(n_batch, n_nodes, D),
            m2.reshape(n_batch, n_edges, D))


# speedup vs baseline: 1.4064x; 1.4064x over previous
import functools

import jax
import jax.numpy as jnp
from jax import lax
from jax.experimental import pallas as pl
from jax.experimental.pallas import tpu as pltpu
from jax.experimental.pallas import tpu_sc as plsc

D = 128
LANES = 16
N_SUBCORES = 16
CHUNK = 128


def _tables_body(x_ref, w1_ref, w2_ref, be_ref, p_ref, q_ref):
    xb = x_ref[...]
    p_ref[...] = (
        jnp.dot(xb, w1_ref[...], preferred_element_type=jnp.float32)
        + be_ref[...]
    )
    q_ref[...] = jnp.dot(xb, w2_ref[...], preferred_element_type=jnp.float32)


def _edge_mm_body(ea_ref, w3_ref, t_ref):
    t_ref[...] = jnp.dot(
        ea_ref[...], w3_ref[...], preferred_element_type=jnp.float32
    )


def _node_body(x_ref, agg_ref, wn_ref, bn_ref, o_ref):
    acc = jnp.dot(x_ref[...], wn_ref[:D, :], preferred_element_type=jnp.float32)
    acc += jnp.dot(agg_ref[...], wn_ref[D:, :], preferred_element_type=jnp.float32)
    o_ref[...] = jnp.maximum(acc + bn_ref[...], 0.0)


def _row_block(bm):
    return pl.BlockSpec((bm, D), lambda i: (i, 0))


def _full_block(shape):
    return pl.BlockSpec(shape, lambda i: tuple(0 for _ in shape))


def _make_sc_edge_kernel(n_nodes, n_edges, n_batch):
    chunks_per_batch = n_edges // CHUNK
    n_iters = (chunks_per_batch + N_SUBCORES - 1) // N_SUBCORES
    rows_per_tile = (n_nodes // N_SUBCORES) // 8 * 8
    rows_rem = n_nodes - rows_per_tile * N_SUBCORES
    groups = D // LANES

    mesh = plsc.VectorSubcoreMesh(core_axis_name="c", subcore_axis_name="s")

    @functools.partial(
        pl.kernel,
        out_type=[
            jax.ShapeDtypeStruct((n_batch * n_edges, D), jnp.float32),
            jax.ShapeDtypeStruct((n_batch * n_nodes, D), jnp.float32),
        ],
        mesh=mesh,
        scratch_types=[
            pltpu.VMEM((CHUNK, D), jnp.float32),
            pltpu.VMEM((CHUNK, D), jnp.float32),
            pltpu.VMEM((CHUNK, D), jnp.float32),
            pltpu.VMEM((CHUNK,), jnp.int32),
            pltpu.VMEM((CHUNK,), jnp.int32),
            pltpu.VMEM((1, CHUNK), jnp.int32),
            pltpu.VMEM_SHARED((n_nodes, D), jnp.float32),
            pltpu.SemaphoreType.DMA,
            pltpu.SemaphoreType.DMA,
            pltpu.SemaphoreType.DMA,
            pltpu.SemaphoreType.DMA,
            pltpu.SemaphoreType.DMA,
        ],
    )
    def sc_kernel(t_hbm, p_hbm, q_hbm, si_hbm, di_hbm, draw_hbm,
                  m_hbm, agg_hbm, tb, g1, g2, sib, dib, scb, agg_sp,
                  sem_t, sem_ix, sem_sc, sem_g, sem_st):
        b = lax.axis_index("c")
        s = lax.axis_index("s")

        @plsc.parallel_loop(0, CHUNK, unroll=4)
        def _(r):
            for k in range(groups):
                tb[r, pl.ds(k * LANES, LANES)] = jnp.zeros((LANES,), jnp.float32)
        zrows = 0
        while zrows < rows_per_tile:
            step = min(CHUNK, rows_per_tile - zrows)
            pltpu.sync_copy(
                tb.at[pl.ds(0, step)],
                agg_sp.at[pl.ds(s * rows_per_tile + zrows, step)],
            )
            zrows += step
        if rows_rem:
            @pl.when(s == N_SUBCORES - 1)
            def _():
                pltpu.sync_copy(
                    tb.at[pl.ds(0, rows_rem)],
                    agg_sp.at[pl.ds(rows_per_tile * N_SUBCORES, rows_rem)],
                )
        plsc.subcore_barrier()

        def chunk_body(t, carry):
            jloc = s + t * N_SUBCORES

            @pl.when(jloc < chunks_per_batch)
            def _():
                j = b * chunks_per_batch + jloc
                row0 = j * CHUNK
                h_t = pltpu.async_copy(t_hbm.at[pl.ds(row0, CHUNK)], tb, sem_t)
                h_si = pltpu.async_copy(si_hbm.at[pl.ds(row0, CHUNK)], sib, sem_ix)
                h_di = pltpu.async_copy(di_hbm.at[pl.ds(row0, CHUNK)], dib, sem_ix)
                h_sc = pltpu.async_copy(
                    draw_hbm.at[pl.ds(row0, CHUNK)], scb.at[0], sem_sc)
                h_si.wait()
                h_di.wait()
                h_g1 = pltpu.async_copy(p_hbm.at[sib], g1, sem_g)
                h_g2 = pltpu.async_copy(q_hbm.at[dib], g2, sem_g)
                h_t.wait()
                h_g1.wait()
                h_g2.wait()

                @plsc.parallel_loop(0, CHUNK, unroll=4)
                def _(r):
                    for k in range(groups):
                        sl = pl.ds(k * LANES, LANES)
                        tb[r, sl] = jnp.maximum(
                            tb[r, sl] + g1[r, sl] + g2[r, sl], 0.0
                        )

                h_m = pltpu.async_copy(tb, m_hbm.at[pl.ds(row0, CHUNK)], sem_st)
                h_sc.wait()
                pltpu.sync_copy(tb, agg_sp.at[scb.at[0]], add=True)
                h_m.wait()
            return carry
        lax.fori_loop(0, n_iters, chunk_body, None)

        plsc.subcore_barrier()
        pltpu.sync_copy(
            agg_sp.at[pl.ds(s * rows_per_tile, rows_per_tile)],
            agg_hbm.at[pl.ds(b * n_nodes + s * rows_per_tile, rows_per_tile)],
        )
        if rows_rem:
            @pl.when(s == N_SUBCORES - 1)
            def _():
                base = rows_per_tile * N_SUBCORES
                pltpu.sync_copy(
                    agg_sp.at[pl.ds(base, rows_rem)],
                    agg_hbm.at[pl.ds(b * n_nodes + base, rows_rem)],
                )

    return sc_kernel


def kernel(x, edge_index, edge_attr, W_edge, b_edge, W_node, b_node):
    n_batch, n_nodes, d = x.shape
    n_edges = edge_index.shape[1]
    assert d == D and n_edges % CHUNK == 0 and n_nodes % N_SUBCORES == 0

    x2 = x.reshape(n_batch * n_nodes, D)
    ea2 = edge_attr.reshape(n_batch * n_edges, D)
    w1 = W_edge[:D]
    w2 = W_edge[D:2 * D]
    w3 = W_edge[2 * D:]
    be = b_edge.reshape(1, D)
    bn = b_node.reshape(1, D)

    src = edge_index[0].astype(jnp.int32)
    dst = edge_index[1].astype(jnp.int32)
    boff = (jnp.arange(n_batch, dtype=jnp.int32) * n_nodes)[:, None]
    src_adj = (src[None, :] + boff).reshape(-1)
    dst_adj = (dst[None, :] + boff).reshape(-1)
    dst_raw = jnp.broadcast_to(dst, (n_batch, n_edges)).reshape(-1)

    bm = 2000
    p2, q2 = pl.pallas_call(
        _tables_body,
        grid=(n_batch * n_nodes // bm,),
        in_specs=[_row_block(bm), _full_block((D, D)), _full_block((D, D)),
                  _full_block((1, D))],
        out_specs=[_row_block(bm), _row_block(bm)],
        out_shape=[jax.ShapeDtypeStruct((n_batch * n_nodes, D), jnp.float32)] * 2,
    )(x2, w1, w2, be)

    bme = 2000
    t2 = pl.pallas_call(
        _edge_mm_body,
        grid=(n_batch * n_edges // bme,),
        in_specs=[_row_block(bme), _full_block((D, D))],
        out_specs=_row_block(bme),
        out_shape=jax.ShapeDtypeStruct((n_batch * n_edges, D), jnp.float32),
    )(ea2, w3)

    sc = _make_sc_edge_kernel(n_nodes, n_edges, n_batch)
    m2, agg2 = sc(t2, p2, q2, src_adj, dst_adj, dst_raw)

    new_x2 = pl.pallas_call(
        _node_body,
        grid=(n_batch * n_nodes // bm,),
        in_specs=[_row_block(bm), _row_block(bm), _full_block((2 * D, D)),
                  _full_block((1, D))],
        out_specs=_row_block(bm),
        out_shape=jax.ShapeDtypeStruct((n_batch * n_nodes, D), jnp.float32),
    )(x2, agg2, W_node, bn)

    return (new_x2.reshape(n_batch, n_nodes, D),
            m2.reshape(n_batch, n_edges, D))


# speedup vs baseline: 1.5384x; 1.0938x over previous
import functools

import jax
import jax.numpy as jnp
from jax import lax
from jax.experimental import pallas as pl
from jax.experimental.pallas import tpu as pltpu
from jax.experimental.pallas import tpu_sc as plsc

D = 128
LANES = 16
N_SUBCORES = 16
CHUNK = 128


def _tables_body(x_ref, w1_ref, w2_ref, be_ref, p_ref, q_ref):
    xb = x_ref[...]
    p_ref[...] = (
        jnp.dot(xb, w1_ref[...], preferred_element_type=jnp.float32)
        + be_ref[...]
    )
    q_ref[...] = jnp.dot(xb, w2_ref[...], preferred_element_type=jnp.float32)


def _edge_mm_body(ea_ref, w3_ref, t_ref):
    t_ref[...] = jnp.dot(
        ea_ref[...], w3_ref[...], preferred_element_type=jnp.float32
    )


def _node_body(x_ref, agg_ref, wn_ref, bn_ref, o_ref):
    acc = jnp.dot(x_ref[...], wn_ref[:D, :], preferred_element_type=jnp.float32)
    acc += jnp.dot(agg_ref[...], wn_ref[D:, :], preferred_element_type=jnp.float32)
    o_ref[...] = jnp.maximum(acc + bn_ref[...], 0.0)


def _row_block(bm):
    return pl.BlockSpec((bm, D), lambda i: (i, 0))


def _full_block(shape):
    return pl.BlockSpec(shape, lambda i: tuple(0 for _ in shape))


def _make_sc_edge_kernel(n_nodes, n_edges, n_batch):
    chunks_per_batch = n_edges // CHUNK
    n_iters = (chunks_per_batch + N_SUBCORES - 1) // N_SUBCORES
    rows_per_tile = (n_nodes // N_SUBCORES) // 8 * 8
    rows_rem = n_nodes - rows_per_tile * N_SUBCORES
    groups = D // LANES

    mesh = plsc.VectorSubcoreMesh(core_axis_name="c", subcore_axis_name="s")

    @functools.partial(
        pl.kernel,
        out_type=[
            jax.ShapeDtypeStruct((n_batch * n_edges, D), jnp.float32),
            jax.ShapeDtypeStruct((n_batch * n_nodes, D), jnp.float32),
        ],
        mesh=mesh,
        scratch_types=[
            pltpu.VMEM((CHUNK, D), jnp.float32),
            pltpu.VMEM((CHUNK, D), jnp.float32),
            pltpu.VMEM((CHUNK, D), jnp.float32),
            pltpu.VMEM((CHUNK,), jnp.int32),
            pltpu.VMEM((CHUNK,), jnp.int32),
            pltpu.VMEM((1, CHUNK), jnp.int32),
            pltpu.VMEM_SHARED((n_nodes, D), jnp.float32),
            pltpu.SemaphoreType.DMA,
            pltpu.SemaphoreType.DMA,
            pltpu.SemaphoreType.DMA,
            pltpu.SemaphoreType.DMA,
            pltpu.SemaphoreType.DMA,
        ],
    )
    def sc_kernel(t_hbm, p_hbm, q_hbm, si_hbm, di_hbm, draw_hbm,
                  m_hbm, agg_hbm, tb, g1, g2, sib, dib, scb, agg_sp,
                  sem_t, sem_ix, sem_sc, sem_g, sem_st):
        b = lax.axis_index("c")
        s = lax.axis_index("s")

        @plsc.parallel_loop(0, CHUNK, unroll=4)
        def _(r):
            for k in range(groups):
                tb[r, pl.ds(k * LANES, LANES)] = jnp.zeros((LANES,), jnp.float32)
        zrows = 0
        while zrows < rows_per_tile:
            step = min(CHUNK, rows_per_tile - zrows)
            pltpu.sync_copy(
                tb.at[pl.ds(0, step)],
                agg_sp.at[pl.ds(s * rows_per_tile + zrows, step)],
            )
            zrows += step
        if rows_rem:
            @pl.when(s == N_SUBCORES - 1)
            def _():
                pltpu.sync_copy(
                    tb.at[pl.ds(0, rows_rem)],
                    agg_sp.at[pl.ds(rows_per_tile * N_SUBCORES, rows_rem)],
                )
        plsc.subcore_barrier()

        def chunk_body(t, carry):
            jloc = s + t * N_SUBCORES

            @pl.when(jloc < chunks_per_batch)
            def _():
                j = b * chunks_per_batch + jloc
                row0 = j * CHUNK
                h_t = pltpu.async_copy(t_hbm.at[pl.ds(row0, CHUNK)], tb, sem_t)
                h_si = pltpu.async_copy(si_hbm.at[pl.ds(row0, CHUNK)], sib, sem_ix)
                h_di = pltpu.async_copy(di_hbm.at[pl.ds(row0, CHUNK)], dib, sem_ix)
                h_sc = pltpu.async_copy(
                    draw_hbm.at[pl.ds(row0, CHUNK)], scb.at[0], sem_sc)
                h_si.wait()
                h_di.wait()
                h_g1 = pltpu.async_copy(p_hbm.at[sib], g1, sem_g)
                h_g2 = pltpu.async_copy(q_hbm.at[dib], g2, sem_g)
                h_t.wait()
                h_g1.wait()
                h_g2.wait()

                @plsc.parallel_loop(0, CHUNK, unroll=4)
                def _(r):
                    for k in range(groups):
                        sl = pl.ds(k * LANES, LANES)
                        tb[r, sl] = jnp.maximum(
                            tb[r, sl] + g1[r, sl] + g2[r, sl], 0.0
                        )

                h_m = pltpu.async_copy(tb, m_hbm.at[pl.ds(row0, CHUNK)], sem_st)
                h_sc.wait()
                pltpu.sync_copy(tb, agg_sp.at[scb.at[0]], add=True)
                h_m.wait()
            return carry
        lax.fori_loop(0, n_iters, chunk_body, None)

        plsc.subcore_barrier()
        pltpu.sync_copy(
            agg_sp.at[pl.ds(s * rows_per_tile, rows_per_tile)],
            agg_hbm.at[pl.ds(b * n_nodes + s * rows_per_tile, rows_per_tile)],
        )
        if rows_rem:
            @pl.when(s == N_SUBCORES - 1)
            def _():
                base = rows_per_tile * N_SUBCORES
                pltpu.sync_copy(
                    agg_sp.at[pl.ds(base, rows_rem)],
                    agg_hbm.at[pl.ds(b * n_nodes + base, rows_rem)],
                )

    return sc_kernel


def kernel(x, edge_index, edge_attr, W_edge, b_edge, W_node, b_node):
    n_batch, n_nodes, d = x.shape
    n_edges = edge_index.shape[1]
    assert d == D and n_edges % CHUNK == 0 and n_nodes % N_SUBCORES == 0

    x2 = x.reshape(n_batch * n_nodes, D)
    ea2 = edge_attr.reshape(n_batch * n_edges, D)
    w1 = W_edge[:D]
    w2 = W_edge[D:2 * D]
    w3 = W_edge[2 * D:]
    be = b_edge.reshape(1, D)
    bn = b_node.reshape(1, D)

    src = edge_index[0].astype(jnp.int32)
    dst = edge_index[1].astype(jnp.int32)
    boff = (jnp.arange(n_batch, dtype=jnp.int32) * n_nodes)[:, None]
    src_adj = (src[None, :] + boff).reshape(-1)
    dst_adj = (dst[None, :] + boff).reshape(-1)
    dst_raw = jnp.broadcast_to(dst, (n_batch, n_edges)).reshape(-1)

    bm = 4000
    p2, q2 = pl.pallas_call(
        _tables_body,
        grid=(n_batch * n_nodes // bm,),
        in_specs=[_row_block(bm), _full_block((D, D)), _full_block((D, D)),
                  _full_block((1, D))],
        out_specs=[_row_block(bm), _row_block(bm)],
        out_shape=[jax.ShapeDtypeStruct((n_batch * n_nodes, D), jnp.float32)] * 2,
    )(x2, w1, w2, be)

    bme = 5000
    t2 = pl.pallas_call(
        _edge_mm_body,
        grid=(n_batch * n_edges // bme,),
        in_specs=[_row_block(bme), _full_block((D, D))],
        out_specs=_row_block(bme),
        out_shape=jax.ShapeDtypeStruct((n_batch * n_edges, D), jnp.float32),
    )(ea2, w3)

    sc = _make_sc_edge_kernel(n_nodes, n_edges, n_batch)
    m2, agg2 = sc(t2, p2, q2, src_adj, dst_adj, dst_raw)

    new_x2 = pl.pallas_call(
        _node_body,
        grid=(n_batch * n_nodes // bm,),
        in_specs=[_row_block(bm), _row_block(bm), _full_block((2 * D, D)),
                  _full_block((1, D))],
        out_specs=_row_block(bm),
        out_shape=jax.ShapeDtypeStruct((n_batch * n_nodes, D), jnp.float32),
    )(x2, agg2, W_node, bn)

    return (new_x2.reshape(n_batch, n_nodes, D),
            m2.reshape(n_batch, n_edges, D))


# speedup vs baseline: 1.5807x; 1.0275x over previous
import functools

import jax
import jax.numpy as jnp
from jax import lax
from jax.experimental import pallas as pl
from jax.experimental.pallas import tpu as pltpu
from jax.experimental.pallas import tpu_sc as plsc

D = 128
LANES = 16
N_SUBCORES = 16
CHUNK = 128


def _tables_body(x_ref, w1_ref, w2_ref, be_ref, p_ref, q_ref):
    xb = x_ref[...]
    p_ref[...] = (
        jnp.dot(xb, w1_ref[...], preferred_element_type=jnp.float32)
        + be_ref[...]
    )
    q_ref[...] = jnp.dot(xb, w2_ref[...], preferred_element_type=jnp.float32)


def _edge_mm_body(ea_ref, w3_ref, t_ref):
    t_ref[...] = jnp.dot(
        ea_ref[...], w3_ref[...], preferred_element_type=jnp.float32
    )


def _node_body(x_ref, agg_ref, wn_ref, bn_ref, o_ref):
    acc = jnp.dot(x_ref[...], wn_ref[:D, :], preferred_element_type=jnp.float32)
    acc += jnp.dot(agg_ref[...], wn_ref[D:, :], preferred_element_type=jnp.float32)
    o_ref[...] = jnp.maximum(acc + bn_ref[...], 0.0)


def _row_block(bm):
    return pl.BlockSpec((bm, D), lambda i: (i, 0))


def _full_block(shape):
    return pl.BlockSpec(shape, lambda i: tuple(0 for _ in shape))


def _make_sc_edge_kernel(n_nodes, n_edges, n_batch):
    chunks_per_batch = n_edges // CHUNK
    n_iters = (chunks_per_batch + N_SUBCORES - 1) // N_SUBCORES
    rows_per_tile = (n_nodes // N_SUBCORES) // 8 * 8
    rows_rem = n_nodes - rows_per_tile * N_SUBCORES
    groups = D // LANES

    mesh = plsc.VectorSubcoreMesh(core_axis_name="c", subcore_axis_name="s")

    @functools.partial(
        pl.kernel,
        out_type=[
            jax.ShapeDtypeStruct((n_batch * n_edges, D), jnp.float32),
            jax.ShapeDtypeStruct((n_batch * n_nodes, D), jnp.float32),
        ],
        mesh=mesh,
        scratch_types=[
            pltpu.VMEM((CHUNK, D), jnp.float32),
            pltpu.VMEM((CHUNK, D), jnp.float32),
            pltpu.VMEM((CHUNK, D), jnp.float32),
            pltpu.VMEM((CHUNK,), jnp.int32),
            pltpu.VMEM((CHUNK,), jnp.int32),
            pltpu.VMEM((1, CHUNK), jnp.int32),
            pltpu.VMEM_SHARED((n_nodes, D), jnp.float32),
            pltpu.SemaphoreType.DMA,
            pltpu.SemaphoreType.DMA,
            pltpu.SemaphoreType.DMA,
            pltpu.SemaphoreType.DMA,
            pltpu.SemaphoreType.DMA,
        ],
    )
    def sc_kernel(t_hbm, p_hbm, q_hbm, si_hbm, di_hbm, draw_hbm,
                  m_hbm, agg_hbm, tb, g1, g2, sib, dib, scb, agg_sp,
                  sem_t, sem_ix, sem_sc, sem_g, sem_st):
        b = lax.axis_index("c")
        s = lax.axis_index("s")

        @plsc.parallel_loop(0, CHUNK, unroll=4)
        def _(r):
            for k in range(groups):
                tb[r, pl.ds(k * LANES, LANES)] = jnp.zeros((LANES,), jnp.float32)
        zrows = 0
        while zrows < rows_per_tile:
            step = min(CHUNK, rows_per_tile - zrows)
            pltpu.sync_copy(
                tb.at[pl.ds(0, step)],
                agg_sp.at[pl.ds(s * rows_per_tile + zrows, step)],
            )
            zrows += step
        if rows_rem:
            @pl.when(s == N_SUBCORES - 1)
            def _():
                pltpu.sync_copy(
                    tb.at[pl.ds(0, rows_rem)],
                    agg_sp.at[pl.ds(rows_per_tile * N_SUBCORES, rows_rem)],
                )
        plsc.subcore_barrier()

        def chunk_body(t, carry):
            jloc = s + t * N_SUBCORES

            @pl.when(jloc < chunks_per_batch)
            def _():
                j = b * chunks_per_batch + jloc
                row0 = j * CHUNK
                h_t = pltpu.async_copy(t_hbm.at[pl.ds(row0, CHUNK)], tb, sem_t)
                h_si = pltpu.async_copy(si_hbm.at[pl.ds(row0, CHUNK)], sib, sem_ix)
                h_di = pltpu.async_copy(di_hbm.at[pl.ds(row0, CHUNK)], dib, sem_ix)
                h_sc = pltpu.async_copy(
                    draw_hbm.at[pl.ds(row0, CHUNK)], scb.at[0], sem_sc)
                h_si.wait()
                h_di.wait()
                h_g1 = pltpu.async_copy(p_hbm.at[sib], g1, sem_g)
                h_g2 = pltpu.async_copy(q_hbm.at[dib], g2, sem_g)
                h_t.wait()
                h_g1.wait()
                h_g2.wait()

                @plsc.parallel_loop(0, CHUNK, unroll=4)
                def _(r):
                    for k in range(groups):
                        sl = pl.ds(k * LANES, LANES)
                        tb[r, sl] = jnp.maximum(
                            tb[r, sl] + g1[r, sl] + g2[r, sl], 0.0
                        )

                h_m = pltpu.async_copy(tb, m_hbm.at[pl.ds(row0, CHUNK)], sem_st)
                h_sc.wait()
                pltpu.sync_copy(tb, agg_sp.at[scb.at[0]], add=True)
                h_m.wait()
            return carry
        lax.fori_loop(0, n_iters, chunk_body, None)

        plsc.subcore_barrier()
        pltpu.sync_copy(
            agg_sp.at[pl.ds(s * rows_per_tile, rows_per_tile)],
            agg_hbm.at[pl.ds(b * n_nodes + s * rows_per_tile, rows_per_tile)],
        )
        if rows_rem:
            @pl.when(s == N_SUBCORES - 1)
            def _():
                base = rows_per_tile * N_SUBCORES
                pltpu.sync_copy(
                    agg_sp.at[pl.ds(base, rows_rem)],
                    agg_hbm.at[pl.ds(b * n_nodes + base, rows_rem)],
                )

    return sc_kernel


def kernel(x, edge_index, edge_attr, W_edge, b_edge, W_node, b_node):
    n_batch, n_nodes, d = x.shape
    n_edges = edge_index.shape[1]
    assert d == D and n_edges % CHUNK == 0 and n_nodes % N_SUBCORES == 0

    x2 = x.reshape(n_batch * n_nodes, D)
    ea2 = edge_attr.reshape(n_batch * n_edges, D)
    w1 = W_edge[:D]
    w2 = W_edge[D:2 * D]
    w3 = W_edge[2 * D:]
    be = b_edge.reshape(1, D)
    bn = b_node.reshape(1, D)

    src = edge_index[0].astype(jnp.int32)
    dst = edge_index[1].astype(jnp.int32)
    boff = (jnp.arange(n_batch, dtype=jnp.int32) * n_nodes)[:, None]
    src_adj = (src[None, :] + boff).reshape(-1)
    dst_adj = (dst[None, :] + boff).reshape(-1)
    dst_raw = jnp.broadcast_to(dst, (n_batch, n_edges)).reshape(-1)

    bm = 10000
    p2, q2 = pl.pallas_call(
        _tables_body,
        grid=(n_batch * n_nodes // bm,),
        in_specs=[_row_block(bm), _full_block((D, D)), _full_block((D, D)),
                  _full_block((1, D))],
        out_specs=[_row_block(bm), _row_block(bm)],
        out_shape=[jax.ShapeDtypeStruct((n_batch * n_nodes, D), jnp.float32)] * 2,
    )(x2, w1, w2, be)

    bme = 10000
    t2 = pl.pallas_call(
        _edge_mm_body,
        grid=(n_batch * n_edges // bme,),
        in_specs=[_row_block(bme), _full_block((D, D))],
        out_specs=_row_block(bme),
        out_shape=jax.ShapeDtypeStruct((n_batch * n_edges, D), jnp.float32),
    )(ea2, w3)

    sc = _make_sc_edge_kernel(n_nodes, n_edges, n_batch)
    m2, agg2 = sc(t2, p2, q2, src_adj, dst_adj, dst_raw)

    new_x2 = pl.pallas_call(
        _node_body,
        grid=(n_batch * n_nodes // bm,),
        in_specs=[_row_block(bm), _row_block(bm), _full_block((2 * D, D)),
                  _full_block((1, D))],
        out_specs=_row_block(bm),
        out_shape=jax.ShapeDtypeStruct((n_batch * n_nodes, D), jnp.float32),
    )(x2, agg2, W_node, bn)

    return (new_x2.reshape(n_batch, n_nodes, D),
            m2.reshape(n_batch, n_edges, D))


# speedup vs baseline: 1.5990x; 1.0116x over previous
import functools

import jax
import jax.numpy as jnp
from jax import lax
from jax.experimental import pallas as pl
from jax.experimental.pallas import tpu as pltpu
from jax.experimental.pallas import tpu_sc as plsc

D = 128
LANES = 16
N_SUBCORES = 16
CHUNK = 128


def _tables_body(x_ref, w1_ref, w2_ref, be_ref, p_ref, q_ref):
    xb = x_ref[...]
    p_ref[...] = (
        jnp.dot(xb, w1_ref[...], preferred_element_type=jnp.float32)
        + be_ref[...]
    )
    q_ref[...] = jnp.dot(xb, w2_ref[...], preferred_element_type=jnp.float32)


def _edge_mm_body(ea_ref, w3_ref, t_ref):
    t_ref[...] = jnp.dot(
        ea_ref[...], w3_ref[...], preferred_element_type=jnp.float32
    )


def _node_body(x_ref, agg_ref, wn_ref, bn_ref, o_ref):
    acc = jnp.dot(x_ref[...], wn_ref[:D, :], preferred_element_type=jnp.float32)
    acc += jnp.dot(agg_ref[...], wn_ref[D:, :], preferred_element_type=jnp.float32)
    o_ref[...] = jnp.maximum(acc + bn_ref[...], 0.0)


def _row_block(bm):
    return pl.BlockSpec((bm, D), lambda i: (i, 0))


def _full_block(shape):
    return pl.BlockSpec(shape, lambda i: tuple(0 for _ in shape))


def _make_sc_edge_kernel(n_nodes, n_edges, n_batch):
    chunks_per_batch = n_edges // CHUNK
    n_iters = (chunks_per_batch + N_SUBCORES - 1) // N_SUBCORES
    rows_per_tile = (n_nodes // N_SUBCORES) // 8 * 8
    rows_rem = n_nodes - rows_per_tile * N_SUBCORES
    groups = D // LANES

    mesh = plsc.VectorSubcoreMesh(core_axis_name="c", subcore_axis_name="s")

    @functools.partial(
        pl.kernel,
        out_type=[
            jax.ShapeDtypeStruct((n_batch * n_edges, D), jnp.float32),
            jax.ShapeDtypeStruct((n_batch * n_nodes, D), jnp.float32),
        ],
        mesh=mesh,
        scratch_types=[
            pltpu.VMEM((CHUNK, D), jnp.float32),
            pltpu.VMEM((CHUNK, D), jnp.float32),
            pltpu.VMEM((CHUNK, D), jnp.float32),
            pltpu.VMEM((CHUNK,), jnp.int32),
            pltpu.VMEM((CHUNK,), jnp.int32),
            pltpu.VMEM((1, CHUNK), jnp.int32),
            pltpu.VMEM_SHARED((n_nodes, D), jnp.float32),
            pltpu.SemaphoreType.DMA,
            pltpu.SemaphoreType.DMA,
            pltpu.SemaphoreType.DMA,
            pltpu.SemaphoreType.DMA,
            pltpu.SemaphoreType.DMA,
        ],
    )
    def sc_kernel(t_hbm, p_hbm, q_hbm, si_hbm, di_hbm, draw_hbm,
                  m_hbm, agg_hbm, tb, g1, g2, sib, dib, scb, agg_sp,
                  sem_t, sem_ix, sem_sc, sem_g, sem_st):
        b = lax.axis_index("c")
        s = lax.axis_index("s")

        @plsc.parallel_loop(0, CHUNK, unroll=4)
        def _(r):
            for k in range(groups):
                tb[r, pl.ds(k * LANES, LANES)] = jnp.zeros((LANES,), jnp.float32)
        zrows = 0
        while zrows < rows_per_tile:
            step = min(CHUNK, rows_per_tile - zrows)
            pltpu.sync_copy(
                tb.at[pl.ds(0, step)],
                agg_sp.at[pl.ds(s * rows_per_tile + zrows, step)],
            )
            zrows += step
        if rows_rem:
            @pl.when(s == N_SUBCORES - 1)
            def _():
                pltpu.sync_copy(
                    tb.at[pl.ds(0, rows_rem)],
                    agg_sp.at[pl.ds(rows_per_tile * N_SUBCORES, rows_rem)],
                )
        plsc.subcore_barrier()

        def chunk_body(t, carry):
            jloc = s + t * N_SUBCORES

            @pl.when(jloc < chunks_per_batch)
            def _():
                j = b * chunks_per_batch + jloc
                row0 = j * CHUNK
                h_t = pltpu.async_copy(t_hbm.at[pl.ds(row0, CHUNK)], tb, sem_t)
                h_si = pltpu.async_copy(si_hbm.at[pl.ds(row0, CHUNK)], sib, sem_ix)
                h_di = pltpu.async_copy(di_hbm.at[pl.ds(row0, CHUNK)], dib, sem_ix)
                h_sc = pltpu.async_copy(
                    draw_hbm.at[pl.ds(row0, CHUNK)], scb.at[0], sem_sc)
                h_si.wait()
                h_di.wait()
                h_g1 = pltpu.async_copy(p_hbm.at[sib], g1, sem_g)
                h_g2 = pltpu.async_copy(q_hbm.at[dib], g2, sem_g)
                h_t.wait()
                h_g1.wait()
                h_g2.wait()

                half = CHUNK // 2

                @plsc.parallel_loop(0, half, unroll=4)
                def _(r):
                    for k in range(groups):
                        sl = pl.ds(k * LANES, LANES)
                        tb[r, sl] = jnp.maximum(
                            tb[r, sl] + g1[r, sl] + g2[r, sl], 0.0
                        )

                h_m0 = pltpu.async_copy(
                    tb.at[pl.ds(0, half)],
                    m_hbm.at[pl.ds(row0, half)], sem_st)

                @plsc.parallel_loop(half, CHUNK, unroll=4)
                def _(r):
                    for k in range(groups):
                        sl = pl.ds(k * LANES, LANES)
                        tb[r, sl] = jnp.maximum(
                            tb[r, sl] + g1[r, sl] + g2[r, sl], 0.0
                        )

                h_m1 = pltpu.async_copy(
                    tb.at[pl.ds(half, half)],
                    m_hbm.at[pl.ds(row0 + half, half)], sem_st)
                h_sc.wait()
                pltpu.sync_copy(tb, agg_sp.at[scb.at[0]], add=True)
                h_m0.wait()
                h_m1.wait()
            return carry
        lax.fori_loop(0, n_iters, chunk_body, None)

        plsc.subcore_barrier()
        pltpu.sync_copy(
            agg_sp.at[pl.ds(s * rows_per_tile, rows_per_tile)],
            agg_hbm.at[pl.ds(b * n_nodes + s * rows_per_tile, rows_per_tile)],
        )
        if rows_rem:
            @pl.when(s == N_SUBCORES - 1)
            def _():
                base = rows_per_tile * N_SUBCORES
                pltpu.sync_copy(
                    agg_sp.at[pl.ds(base, rows_rem)],
                    agg_hbm.at[pl.ds(b * n_nodes + base, rows_rem)],
                )

    return sc_kernel


def kernel(x, edge_index, edge_attr, W_edge, b_edge, W_node, b_node):
    n_batch, n_nodes, d = x.shape
    n_edges = edge_index.shape[1]
    assert d == D and n_edges % CHUNK == 0 and n_nodes % N_SUBCORES == 0

    x2 = x.reshape(n_batch * n_nodes, D)
    ea2 = edge_attr.reshape(n_batch * n_edges, D)
    w1 = W_edge[:D]
    w2 = W_edge[D:2 * D]
    w3 = W_edge[2 * D:]
    be = b_edge.reshape(1, D)
    bn = b_node.reshape(1, D)

    src = edge_index[0].astype(jnp.int32)
    dst = edge_index[1].astype(jnp.int32)
    boff = (jnp.arange(n_batch, dtype=jnp.int32) * n_nodes)[:, None]
    src_adj = (src[None, :] + boff).reshape(-1)
    dst_adj = (dst[None, :] + boff).reshape(-1)
    dst_raw = jnp.broadcast_to(dst, (n_batch, n_edges)).reshape(-1)

    bm = 10000
    p2, q2 = pl.pallas_call(
        _tables_body,
        grid=(n_batch * n_nodes // bm,),
        in_specs=[_row_block(bm), _full_block((D, D)), _full_block((D, D)),
                  _full_block((1, D))],
        out_specs=[_row_block(bm), _row_block(bm)],
        out_shape=[jax.ShapeDtypeStruct((n_batch * n_nodes, D), jnp.float32)] * 2,
    )(x2, w1, w2, be)

    bme = 10000
    t2 = pl.pallas_call(
        _edge_mm_body,
        grid=(n_batch * n_edges // bme,),
        in_specs=[_row_block(bme), _full_block((D, D))],
        out_specs=_row_block(bme),
        out_shape=jax.ShapeDtypeStruct((n_batch * n_edges, D), jnp.float32),
    )(ea2, w3)

    sc = _make_sc_edge_kernel(n_nodes, n_edges, n_batch)
    m2, agg2 = sc(t2, p2, q2, src_adj, dst_adj, dst_raw)

    new_x2 = pl.pallas_call(
        _node_body,
        grid=(n_batch * n_nodes // bm,),
        in_specs=[_row_block(bm), _row_block(bm), _full_block((2 * D, D)),
                  _full_block((1, D))],
        out_specs=_row_block(bm),
        out_shape=jax.ShapeDtypeStruct((n_batch * n_nodes, D), jnp.float32),
    )(x2, agg2, W_node, bn)

    return (new_x2.reshape(n_batch, n_nodes, D),
            m2.reshape(n_batch, n_edges, D))


# speedup vs baseline: 1.6160x; 1.0106x over previous
import functools

import jax
import jax.numpy as jnp
from jax import lax
from jax.experimental import pallas as pl
from jax.experimental.pallas import tpu as pltpu
from jax.experimental.pallas import tpu_sc as plsc

D = 128
LANES = 16
N_SUBCORES = 16
CHUNK = 128


def _tables_body(x_ref, w1_ref, w2_ref, be_ref, p_ref, q_ref):
    xb = x_ref[...]
    p_ref[...] = (
        jnp.dot(xb, w1_ref[...], preferred_element_type=jnp.float32)
        + be_ref[...]
    )
    q_ref[...] = jnp.dot(xb, w2_ref[...], preferred_element_type=jnp.float32)


def _edge_mm_body(ea_ref, w3_ref, t_ref):
    t_ref[...] = jnp.dot(
        ea_ref[...], w3_ref[...], preferred_element_type=jnp.float32
    )


def _node_body(x_ref, agg_ref, wn_ref, bn_ref, o_ref):
    acc = jnp.dot(x_ref[...], wn_ref[:D, :], preferred_element_type=jnp.float32)
    acc += jnp.dot(agg_ref[...], wn_ref[D:, :], preferred_element_type=jnp.float32)
    o_ref[...] = jnp.maximum(acc + bn_ref[...], 0.0)


def _row_block(bm):
    return pl.BlockSpec((bm, D), lambda i: (i, 0))


def _full_block(shape):
    return pl.BlockSpec(shape, lambda i: tuple(0 for _ in shape))


def _make_sc_edge_kernel(n_nodes, n_edges, n_batch):
    chunks_per_batch = n_edges // CHUNK
    n_iters = (chunks_per_batch + N_SUBCORES - 1) // N_SUBCORES
    rows_per_tile = (n_nodes // N_SUBCORES) // 8 * 8
    rows_rem = n_nodes - rows_per_tile * N_SUBCORES
    groups = D // LANES

    mesh = plsc.VectorSubcoreMesh(core_axis_name="c", subcore_axis_name="s")

    @functools.partial(
        pl.kernel,
        out_type=[
            jax.ShapeDtypeStruct((n_batch * n_edges, D), jnp.float32),
            jax.ShapeDtypeStruct((n_batch * n_nodes, D), jnp.float32),
        ],
        mesh=mesh,
        scratch_types=[
            pltpu.VMEM((CHUNK, D), jnp.float32),
            pltpu.VMEM((CHUNK, D), jnp.float32),
            pltpu.VMEM((CHUNK, D), jnp.float32),
            pltpu.VMEM((CHUNK,), jnp.int32),
            pltpu.VMEM((CHUNK,), jnp.int32),
            pltpu.VMEM((1, CHUNK), jnp.int32),
            pltpu.VMEM_SHARED((n_nodes, D), jnp.float32),
            pltpu.SemaphoreType.DMA,
            pltpu.SemaphoreType.DMA,
            pltpu.SemaphoreType.DMA,
            pltpu.SemaphoreType.DMA,
            pltpu.SemaphoreType.DMA,
            pltpu.SemaphoreType.DMA,
        ],
    )
    def sc_kernel(t_hbm, p_hbm, q_hbm, si_hbm, di_hbm, draw_hbm,
                  m_hbm, agg_hbm, tb, g1, g2, sib, dib, scb, agg_sp,
                  sem_t, sem_ix, sem_sc, sem_ga, sem_gb, sem_st):
        b = lax.axis_index("c")
        s = lax.axis_index("s")

        @plsc.parallel_loop(0, CHUNK, unroll=4)
        def _(r):
            for k in range(groups):
                tb[r, pl.ds(k * LANES, LANES)] = jnp.zeros((LANES,), jnp.float32)
        zrows = 0
        while zrows < rows_per_tile:
            step = min(CHUNK, rows_per_tile - zrows)
            pltpu.sync_copy(
                tb.at[pl.ds(0, step)],
                agg_sp.at[pl.ds(s * rows_per_tile + zrows, step)],
            )
            zrows += step
        if rows_rem:
            @pl.when(s == N_SUBCORES - 1)
            def _():
                pltpu.sync_copy(
                    tb.at[pl.ds(0, rows_rem)],
                    agg_sp.at[pl.ds(rows_per_tile * N_SUBCORES, rows_rem)],
                )
        plsc.subcore_barrier()

        def chunk_body(t, carry):
            jloc = s + t * N_SUBCORES

            @pl.when(jloc < chunks_per_batch)
            def _():
                j = b * chunks_per_batch + jloc
                row0 = j * CHUNK
                h_t = pltpu.async_copy(t_hbm.at[pl.ds(row0, CHUNK)], tb, sem_t)
                h_si = pltpu.async_copy(si_hbm.at[pl.ds(row0, CHUNK)], sib, sem_ix)
                h_di = pltpu.async_copy(di_hbm.at[pl.ds(row0, CHUNK)], dib, sem_ix)
                h_sc = pltpu.async_copy(
                    draw_hbm.at[pl.ds(row0, CHUNK)], scb.at[0], sem_sc)
                half = CHUNK // 2
                h_si.wait()
                h_di.wait()
                h_g1a = pltpu.async_copy(
                    p_hbm.at[sib.at[pl.ds(0, half)]],
                    g1.at[pl.ds(0, half)], sem_ga)
                h_g2a = pltpu.async_copy(
                    q_hbm.at[dib.at[pl.ds(0, half)]],
                    g2.at[pl.ds(0, half)], sem_ga)
                h_g1b = pltpu.async_copy(
                    p_hbm.at[sib.at[pl.ds(half, half)]],
                    g1.at[pl.ds(half, half)], sem_gb)
                h_g2b = pltpu.async_copy(
                    q_hbm.at[dib.at[pl.ds(half, half)]],
                    g2.at[pl.ds(half, half)], sem_gb)
                h_t.wait()
                h_g1a.wait()
                h_g2a.wait()

                @plsc.parallel_loop(0, half, unroll=4)
                def _(r):
                    for k in range(groups):
                        sl = pl.ds(k * LANES, LANES)
                        tb[r, sl] = jnp.maximum(
                            tb[r, sl] + g1[r, sl] + g2[r, sl], 0.0
                        )

                h_m0 = pltpu.async_copy(
                    tb.at[pl.ds(0, half)],
                    m_hbm.at[pl.ds(row0, half)], sem_st)
                h_g1b.wait()
                h_g2b.wait()

                @plsc.parallel_loop(half, CHUNK, unroll=4)
                def _(r):
                    for k in range(groups):
                        sl = pl.ds(k * LANES, LANES)
                        tb[r, sl] = jnp.maximum(
                            tb[r, sl] + g1[r, sl] + g2[r, sl], 0.0
                        )

                h_m1 = pltpu.async_copy(
                    tb.at[pl.ds(half, half)],
                    m_hbm.at[pl.ds(row0 + half, half)], sem_st)
                h_sc.wait()
                pltpu.sync_copy(tb, agg_sp.at[scb.at[0]], add=True)
                h_m0.wait()
                h_m1.wait()
            return carry
        lax.fori_loop(0, n_iters, chunk_body, None)

        plsc.subcore_barrier()
        pltpu.sync_copy(
            agg_sp.at[pl.ds(s * rows_per_tile, rows_per_tile)],
            agg_hbm.at[pl.ds(b * n_nodes + s * rows_per_tile, rows_per_tile)],
        )
        if rows_rem:
            @pl.when(s == N_SUBCORES - 1)
            def _():
                base = rows_per_tile * N_SUBCORES
                pltpu.sync_copy(
                    agg_sp.at[pl.ds(base, rows_rem)],
                    agg_hbm.at[pl.ds(b * n_nodes + base, rows_rem)],
                )

    return sc_kernel


def kernel(x, edge_index, edge_attr, W_edge, b_edge, W_node, b_node):
    n_batch, n_nodes, d = x.shape
    n_edges = edge_index.shape[1]
    assert d == D and n_edges % CHUNK == 0 and n_nodes % N_SUBCORES == 0

    x2 = x.reshape(n_batch * n_nodes, D)
    ea2 = edge_attr.reshape(n_batch * n_edges, D)
    w1 = W_edge[:D]
    w2 = W_edge[D:2 * D]
    w3 = W_edge[2 * D:]
    be = b_edge.reshape(1, D)
    bn = b_node.reshape(1, D)

    src = edge_index[0].astype(jnp.int32)
    dst = edge_index[1].astype(jnp.int32)
    boff = (jnp.arange(n_batch, dtype=jnp.int32) * n_nodes)[:, None]
    src_adj = (src[None, :] + boff).reshape(-1)
    dst_adj = (dst[None, :] + boff).reshape(-1)
    dst_raw = jnp.broadcast_to(dst, (n_batch, n_edges)).reshape(-1)

    bm = 10000
    p2, q2 = pl.pallas_call(
        _tables_body,
        grid=(n_batch * n_nodes // bm,),
        in_specs=[_row_block(bm), _full_block((D, D)), _full_block((D, D)),
                  _full_block((1, D))],
        out_specs=[_row_block(bm), _row_block(bm)],
        out_shape=[jax.ShapeDtypeStruct((n_batch * n_nodes, D), jnp.float32)] * 2,
    )(x2, w1, w2, be)

    bme = 10000
    t2 = pl.pallas_call(
        _edge_mm_body,
        grid=(n_batch * n_edges // bme,),
        in_specs=[_row_block(bme), _full_block((D, D))],
        out_specs=_row_block(bme),
        out_shape=jax.ShapeDtypeStruct((n_batch * n_edges, D), jnp.float32),
    )(ea2, w3)

    sc = _make_sc_edge_kernel(n_nodes, n_edges, n_batch)
    m2, agg2 = sc(t2, p2, q2, src_adj, dst_adj, dst_raw)

    new_x2 = pl.pallas_call(
        _node_body,
        grid=(n_batch * n_nodes // bm,),
        in_specs=[_row_block(bm), _row_block(bm), _full_block((2 * D, D)),
                  _full_block((1, D))],
        out_specs=_row_block(bm),
        out_shape=jax.ShapeDtypeStruct((n_batch * n_nodes, D), jnp.float32),
    )(x2, agg2, W_node, bn)

    return (new_x2.reshape(n_batch, n_nodes, D),
            m2.reshape(n_batch, n_edges, D))
